# baseline (speedup 1.0000x reference)
import functools

import jax
import jax.numpy as jnp
from jax import lax
from jax.experimental import pallas as pl
from jax.experimental.pallas import tpu as pltpu
from jax.experimental.pallas import tpu_sc as plsc

_EMBED_DIM = 16
_INPUT_DIM = 64
_HIDDEN = 64
_BATCH = 16384
_TABLE_ELEMS = 100000 * _EMBED_DIM

_NC = 2
_NS = 16
_NW = _NC * _NS
_BPW = _BATCH // _NW
_CHUNK = 128
_NCHUNK = _BPW // _CHUNK
_L = 16
_EPW = _BPW * _EMBED_DIM
_NE = _EPW // _CHUNK


def _sc_gather(table1d, idx3):
    mesh = plsc.VectorSubcoreMesh(core_axis_name="c", subcore_axis_name="s")

    @functools.partial(
        pl.kernel,
        mesh=mesh,
        compiler_params=pltpu.CompilerParams(use_tc_tiling_on_sc=False,
                                             needs_layout_passes=False),
        out_type=jax.ShapeDtypeStruct((_BATCH, 128), jnp.float32),
        scratch_types=[
            pltpu.VMEM((_NCHUNK, _CHUNK), jnp.int32),
            pltpu.VMEM((_NE, _CHUNK), jnp.int32),
            pltpu.VMEM((_NE, _CHUNK), jnp.float32),
            pltpu.VMEM((_BPW, _EMBED_DIM), jnp.float32),
            pltpu.SemaphoreType.DMA,
        ],
    )
    def gather_kernel(table_hbm, idx_hbm, out_hbm, idx_v, eidx_v, slab_v,
                      rows_v, sem):
        wid = lax.axis_index("s") * _NC + lax.axis_index("c")
        pltpu.sync_copy(idx_hbm.at[wid], idx_v)
        lane = lax.iota(jnp.int32, _L)
        lhi = lax.shift_right_logical(lane, 3)
        lo16 = (lane & 7) * _L
        for g in range(_BPW // _L):
            v = idx_v[g // 8, pl.ds((g % 8) * _L, _L)]
            w = v * _EMBED_DIM
            m_vec = lhi + 2 * g
            for j in range(_EMBED_DIM):
                plsc.store_scatter(eidx_v, [m_vec, lo16 + j], w + j)
        copies = [
            pltpu.async_copy(table_hbm.at[eidx_v.at[m]], slab_v.at[m], sem)
            for m in range(_NE)
        ]
        for cp in copies:
            cp.wait()
        for m in range(_NE):
            for t in range(_CHUNK // _L):
                rows_v[m * 8 + t, :] = slab_v[m, pl.ds(t * _L, _L)]
        pltpu.sync_copy(
            rows_v,
            out_hbm.at[pl.ds(wid * _BPW, _BPW), pl.ds(0, _EMBED_DIM)])

    return gather_kernel(table1d, idx3)


_BR = 2048


def _mlp_body(emb_hbm, feat_ref, w1_ref, b1_ref, w2t_ref, b2_ref, out_ref,
              emb_vmem, sem):
    i = pl.program_id(0)
    cp = pltpu.make_async_copy(emb_hbm.at[pl.ds(i * _BR, _BR), :], emb_vmem,
                               sem)
    cp.start()
    w1e = w1_ref[0:_EMBED_DIM, :]
    w1f = w1_ref[_EMBED_DIM:, :]
    x = jnp.dot(feat_ref[...], w1f, preferred_element_type=jnp.float32)
    cp.wait()
    x = x + jnp.dot(emb_vmem[:, :_EMBED_DIM], w1e,
                    preferred_element_type=jnp.float32)
    h = jnp.maximum(x + b1_ref[...], 0.0)
    out_ref[...] = (jnp.sum(h * w2t_ref[...], axis=1, keepdims=True)
                    + b2_ref[...])


def _tc_mlp(emb128, features, W1, b1r, w2t, b2r, interpret=False):
    grid = (_BATCH // _BR,)
    return pl.pallas_call(
        _mlp_body,
        grid=grid,
        in_specs=[
            pl.BlockSpec(memory_space=pl.ANY),
            pl.BlockSpec((_BR, _INPUT_DIM), lambda i: (i, 0)),
            pl.BlockSpec((_EMBED_DIM + _INPUT_DIM, _HIDDEN), lambda i: (0, 0)),
            pl.BlockSpec((1, _HIDDEN), lambda i: (0, 0)),
            pl.BlockSpec((1, _HIDDEN), lambda i: (0, 0)),
            pl.BlockSpec((1, 1), lambda i: (0, 0)),
        ],
        out_specs=pl.BlockSpec((_BR, 1), lambda i: (i, 0)),
        out_shape=jax.ShapeDtypeStruct((_BATCH, 1), jnp.float32),
        scratch_shapes=[
            pltpu.VMEM((_BR, 128), jnp.float32),
            pltpu.SemaphoreType.DMA,
        ],
        interpret=interpret,
    )(emb128, features, W1, b1r, w2t, b2r)


def kernel(pitcher_id, features, table, W1, b1, W2, b2):
    pid = pitcher_id.astype(jnp.int32)
    idx3 = pid.reshape(_NW, _NCHUNK, _CHUNK)
    emb128 = _sc_gather(table.reshape(_TABLE_ELEMS), idx3)
    b1r = b1.reshape(1, _HIDDEN)
    w2t = W2.reshape(1, _HIDDEN)
    b2r = b2.reshape(1, 1)
    return _tc_mlp(emb128, features, W1, b1r, w2t, b2r)


# speedup vs baseline: 2.1787x; 2.1787x over previous
import functools

import jax
import jax.numpy as jnp
from jax import lax
from jax.experimental import pallas as pl
from jax.experimental.pallas import tpu as pltpu
from jax.experimental.pallas import tpu_sc as plsc

_EMBED_DIM = 16
_INPUT_DIM = 64
_HIDDEN = 64
_BATCH = 16384

_NC = 2
_NS = 16
_NW = _NC * _NS
_BPW = _BATCH // _NW
_CHUNK = 128
_NCHUNK = _BPW // _CHUNK


def _sc_gather(tableT, idx3):
    mesh = plsc.VectorSubcoreMesh(core_axis_name="c", subcore_axis_name="s")

    @functools.partial(
        pl.kernel,
        mesh=mesh,
        compiler_params=pltpu.CompilerParams(use_tc_tiling_on_sc=False,
                                             needs_layout_passes=False),
        out_type=jax.ShapeDtypeStruct((_EMBED_DIM, _BATCH), jnp.float32),
        scratch_types=[
            pltpu.VMEM((_NCHUNK, _CHUNK), jnp.int32),
            pltpu.VMEM((_EMBED_DIM, _BPW), jnp.float32),
            pltpu.SemaphoreType.DMA,
        ],
    )
    def gather_kernel(table_hbm, idx_hbm, out_hbm, idx_v, slab_v, sem):
        wid = lax.axis_index("s") * _NC + lax.axis_index("c")
        pltpu.sync_copy(idx_hbm.at[wid], idx_v)
        copies = [
            pltpu.async_copy(
                table_hbm.at[j].at[idx_v.at[c]],
                slab_v.at[j, pl.ds(c * _CHUNK, _CHUNK)],
                sem,
            )
            for j in range(_EMBED_DIM)
            for c in range(_NCHUNK)
        ]
        for cp in copies:
            cp.wait()
        pltpu.sync_copy(slab_v, out_hbm.at[:, pl.ds(wid * _BPW, _BPW)])

    return gather_kernel(tableT, idx3)


_BC = 2048


def _mlp_body(embT_ref, featT_ref, w1T_ref, b1c_ref, w2c_ref, b2_ref,
              out_ref):
    w1eT = w1T_ref[:, 0:_EMBED_DIM]
    w1fT = w1T_ref[:, _EMBED_DIM:]
    xT = jnp.dot(w1fT, featT_ref[...], preferred_element_type=jnp.float32)
    xT = xT + jnp.dot(w1eT, embT_ref[...], preferred_element_type=jnp.float32)
    hT = jnp.maximum(xT + b1c_ref[...], 0.0)
    o = jnp.sum(hT * w2c_ref[...], axis=0) + b2_ref[0, 0]
    out_ref[...] = jnp.concatenate(
        [o.reshape(1, _BC), jnp.zeros((7, _BC), jnp.float32)], axis=0)


def _tc_mlp(embT, featT, w1T, b1c, w2c, b2r, interpret=False):
    grid = (_BATCH // _BC,)
    return pl.pallas_call(
        _mlp_body,
        grid=grid,
        in_specs=[
            pl.BlockSpec((_EMBED_DIM, _BC), lambda i: (0, i)),
            pl.BlockSpec((_INPUT_DIM, _BC), lambda i: (0, i)),
            pl.BlockSpec((_INPUT_DIM, _EMBED_DIM + _INPUT_DIM),
                         lambda i: (0, 0)),
            pl.BlockSpec((_HIDDEN, 1), lambda i: (0, 0)),
            pl.BlockSpec((_HIDDEN, 1), lambda i: (0, 0)),
            pl.BlockSpec((1, 1), lambda i: (0, 0)),
        ],
        out_specs=pl.BlockSpec((8, _BC), lambda i: (0, i)),
        out_shape=jax.ShapeDtypeStruct((8, _BATCH), jnp.float32),
        interpret=interpret,
    )(embT, featT, w1T, b1c, w2c, b2r)


def kernel(pitcher_id, features, table, W1, b1, W2, b2):
    pid = pitcher_id.astype(jnp.int32)
    idx3 = pid.reshape(_NW, _NCHUNK, _CHUNK)
    embT = _sc_gather(table.T, idx3)
    out8 = _tc_mlp(embT, features.T, W1.T, b1.reshape(_HIDDEN, 1), W2,
                   b2.reshape(1, 1))
    return out8[:1, :].reshape(_BATCH, 1)


# speedup vs baseline: 2.2766x; 1.0449x over previous
import functools

import jax
import jax.numpy as jnp
from jax import lax
from jax.experimental import pallas as pl
from jax.experimental.pallas import tpu as pltpu
from jax.experimental.pallas import tpu_sc as plsc

_EMBED_DIM = 16
_INPUT_DIM = 64
_HIDDEN = 64
_BATCH = 16384
_NT = _BATCH // 128

_NC = 2
_NS = 16
_NW = _NC * _NS
_BPW = _BATCH // _NW
_CHUNK = 128
_NCHUNK = _BPW // _CHUNK


def _sc_gather(tableT, idx3):
    mesh = plsc.VectorSubcoreMesh(core_axis_name="c", subcore_axis_name="s")

    @functools.partial(
        pl.kernel,
        mesh=mesh,
        compiler_params=pltpu.CompilerParams(use_tc_tiling_on_sc=False,
                                             needs_layout_passes=False),
        out_type=jax.ShapeDtypeStruct((_NT, _EMBED_DIM, _CHUNK), jnp.float32),
        scratch_types=[
            pltpu.VMEM((_NCHUNK, _CHUNK), jnp.int32),
            pltpu.VMEM((_NCHUNK, _EMBED_DIM, _CHUNK), jnp.float32),
            pltpu.SemaphoreType.DMA,
        ],
    )
    def gather_kernel(table_hbm, idx_hbm, out_hbm, idx_v, slab_v, sem):
        wid = lax.axis_index("s") * _NC + lax.axis_index("c")
        pltpu.sync_copy(idx_hbm.at[wid], idx_v)
        copies = [
            pltpu.async_copy(
                table_hbm.at[j].at[idx_v.at[c]],
                slab_v.at[c, j],
                sem,
            )
            for j in range(_EMBED_DIM)
            for c in range(_NCHUNK)
        ]
        for cp in copies:
            cp.wait()
        pltpu.sync_copy(slab_v, out_hbm.at[pl.ds(wid * _NCHUNK, _NCHUNK)])

    return gather_kernel(tableT, idx3)


_BC = 2048
_TPB = _BC // 128


def _mlp1_body(featT_ref, w1T_ref, b1c_ref, xf_ref):
    w1fT = w1T_ref[:, _EMBED_DIM:]
    xf_ref[...] = jnp.dot(
        w1fT, featT_ref[...],
        preferred_element_type=jnp.float32) + b1c_ref[...]


def _tc_mlp1(featT, w1T, b1c):
    grid = (_BATCH // _BC,)
    return pl.pallas_call(
        _mlp1_body,
        grid=grid,
        in_specs=[
            pl.BlockSpec((_INPUT_DIM, _BC), lambda i: (0, i)),
            pl.BlockSpec((_INPUT_DIM, _EMBED_DIM + _INPUT_DIM),
                         lambda i: (0, 0)),
            pl.BlockSpec((_HIDDEN, 1), lambda i: (0, 0)),
        ],
        out_specs=pl.BlockSpec((_HIDDEN, _BC), lambda i: (0, i)),
        out_shape=jax.ShapeDtypeStruct((_HIDDEN, _BATCH), jnp.float32),
    )(featT, w1T, b1c)


def _mlp2_body(xf_ref, e_ref, w1T_ref, w2c_ref, b2_ref, out_ref):
    w1eT = w1T_ref[:, 0:_EMBED_DIM]
    e = e_ref[...]
    xe = jnp.concatenate(
        [jnp.dot(w1eT, e[t], preferred_element_type=jnp.float32)
         for t in range(_TPB)], axis=1)
    hT = jnp.maximum(xf_ref[...] + xe, 0.0)
    o = jnp.sum(hT * w2c_ref[...], axis=0) + b2_ref[0, 0]
    out_ref[...] = jnp.concatenate(
        [o.reshape(1, _BC), jnp.zeros((7, _BC), jnp.float32)], axis=0)


def _tc_mlp2(xfT, E, w1T, w2c, b2r):
    grid = (_BATCH // _BC,)
    return pl.pallas_call(
        _mlp2_body,
        grid=grid,
        in_specs=[
            pl.BlockSpec((_HIDDEN, _BC), lambda i: (0, i)),
            pl.BlockSpec((_TPB, _EMBED_DIM, _CHUNK), lambda i: (i, 0, 0)),
            pl.BlockSpec((_INPUT_DIM, _EMBED_DIM + _INPUT_DIM),
                         lambda i: (0, 0)),
            pl.BlockSpec((_HIDDEN, 1), lambda i: (0, 0)),
            pl.BlockSpec((1, 1), lambda i: (0, 0)),
        ],
        out_specs=pl.BlockSpec((8, _BC), lambda i: (0, i)),
        out_shape=jax.ShapeDtypeStruct((8, _BATCH), jnp.float32),
    )(xfT, E, w1T, w2c, b2r)


def kernel(pitcher_id, features, table, W1, b1, W2, b2):
    pid = pitcher_id.astype(jnp.int32)
    idx3 = pid.reshape(_NW, _NCHUNK, _CHUNK)
    w1T = W1.T
    E = _sc_gather(table.T, idx3)
    xfT = _tc_mlp1(features.T, w1T, b1.reshape(_HIDDEN, 1))
    out8 = _tc_mlp2(xfT, E, w1T, W2, b2.reshape(1, 1))
    return out8[:1, :].reshape(_BATCH, 1)


# speedup vs baseline: 2.3132x; 1.0161x over previous
import functools

import jax
import jax.numpy as jnp
from jax import lax
from jax.experimental import pallas as pl
from jax.experimental.pallas import tpu as pltpu
from jax.experimental.pallas import tpu_sc as plsc

_EMBED_DIM = 16
_INPUT_DIM = 64
_HIDDEN = 64
_BATCH = 16384
_NT = _BATCH // 128

_NC = 2
_NS = 16
_NW = _NC * _NS
_BPW = _BATCH // _NW
_CHUNK = 128
_NCHUNK = _BPW // _CHUNK


def _sc_gather(tableT, idx3):
    mesh = plsc.VectorSubcoreMesh(core_axis_name="c", subcore_axis_name="s")

    @functools.partial(
        pl.kernel,
        mesh=mesh,
        compiler_params=pltpu.CompilerParams(use_tc_tiling_on_sc=False,
                                             needs_layout_passes=False),
        out_type=jax.ShapeDtypeStruct((_NT, _EMBED_DIM, _CHUNK), jnp.float32),
        scratch_types=[
            pltpu.VMEM((_NCHUNK, _CHUNK), jnp.int32),
            pltpu.VMEM((_NCHUNK, _EMBED_DIM, _CHUNK), jnp.float32),
            pltpu.SemaphoreType.DMA,
        ],
    )
    def gather_kernel(table_hbm, idx_hbm, out_hbm, idx_v, slab_v, sem):
        wid = lax.axis_index("s") * _NC + lax.axis_index("c")
        pltpu.sync_copy(idx_hbm.at[wid], idx_v)
        copies = [
            pltpu.async_copy(
                table_hbm.at[j].at[idx_v.at[c]],
                slab_v.at[c, j],
                sem,
            )
            for j in range(_EMBED_DIM)
            for c in range(_NCHUNK)
        ]
        for cp in copies:
            cp.wait()
        pltpu.sync_copy(slab_v, out_hbm.at[pl.ds(wid * _NCHUNK, _NCHUNK)])

    return gather_kernel(tableT, idx3)


_BC = 2048
_TPB = _BC // 128


def _mlp1_body(featT_ref, w1T_ref, b1c_ref, xf_ref):
    w1fT = w1T_ref[:, _EMBED_DIM:]
    xf = jnp.dot(w1fT, featT_ref[...],
                 preferred_element_type=jnp.float32) + b1c_ref[...]
    xf_ref[...] = xf.astype(jnp.bfloat16)


def _tc_mlp1(featT, w1T, b1c):
    grid = (_BATCH // _BC,)
    return pl.pallas_call(
        _mlp1_body,
        grid=grid,
        in_specs=[
            pl.BlockSpec((_INPUT_DIM, _BC), lambda i: (0, i)),
            pl.BlockSpec((_INPUT_DIM, _EMBED_DIM + _INPUT_DIM),
                         lambda i: (0, 0)),
            pl.BlockSpec((_HIDDEN, 1), lambda i: (0, 0)),
        ],
        out_specs=pl.BlockSpec((_HIDDEN, _BC), lambda i: (0, i)),
        out_shape=jax.ShapeDtypeStruct((_HIDDEN, _BATCH), jnp.bfloat16),
    )(featT, w1T, b1c)


def _mlp2_body(xf_ref, e_ref, w1T_ref, w2c_ref, b2_ref, out_ref):
    w1eT = w1T_ref[:, 0:_EMBED_DIM]
    e = e_ref[...]
    xe = jnp.concatenate(
        [jnp.dot(w1eT, e[t], preferred_element_type=jnp.float32)
         for t in range(_TPB)], axis=1)
    hT = jnp.maximum(xf_ref[...].astype(jnp.float32) + xe, 0.0)
    o = jnp.sum(hT * w2c_ref[...], axis=0) + b2_ref[0, 0]
    out_ref[...] = jnp.concatenate(
        [o.reshape(1, _BC), jnp.zeros((7, _BC), jnp.float32)], axis=0)


def _tc_mlp2(xfT, E, w1T, w2c, b2r):
    grid = (_BATCH // _BC,)
    return pl.pallas_call(
        _mlp2_body,
        grid=grid,
        in_specs=[
            pl.BlockSpec((_HIDDEN, _BC), lambda i: (0, i)),
            pl.BlockSpec((_TPB, _EMBED_DIM, _CHUNK), lambda i: (i, 0, 0)),
            pl.BlockSpec((_INPUT_DIM, _EMBED_DIM + _INPUT_DIM),
                         lambda i: (0, 0)),
            pl.BlockSpec((_HIDDEN, 1), lambda i: (0, 0)),
            pl.BlockSpec((1, 1), lambda i: (0, 0)),
        ],
        out_specs=pl.BlockSpec((8, _BC), lambda i: (0, i)),
        out_shape=jax.ShapeDtypeStruct((8, _BATCH), jnp.float32),
    )(xfT, E, w1T, w2c, b2r)


def kernel(pitcher_id, features, table, W1, b1, W2, b2):
    pid = pitcher_id.astype(jnp.int32)
    idx3 = pid.reshape(_NW, _NCHUNK, _CHUNK)
    w1T = W1.T
    E = _sc_gather(table.T, idx3)
    xfT = _tc_mlp1(features.T, w1T, b1.reshape(_HIDDEN, 1))
    out8 = _tc_mlp2(xfT, E, w1T, W2, b2.reshape(1, 1))
    return out8[:1, :].reshape(_BATCH, 1)


# speedup vs baseline: 2.3633x; 1.0217x over previous
import functools

import jax
import jax.numpy as jnp
from jax import lax
from jax.experimental import pallas as pl
from jax.experimental.pallas import tpu as pltpu
from jax.experimental.pallas import tpu_sc as plsc

_EMBED_DIM = 16
_INPUT_DIM = 64
_HIDDEN = 64
_BATCH = 16384
_NT = _BATCH // 128

_NC = 2
_NS = 16
_NW = _NC * _NS
_BPW = _BATCH // _NW
_CHUNK = 128
_NCHUNK = _BPW // _CHUNK


def _sc_gather(tableT, idx3):
    mesh = plsc.VectorSubcoreMesh(core_axis_name="c", subcore_axis_name="s")

    @functools.partial(
        pl.kernel,
        mesh=mesh,
        compiler_params=pltpu.CompilerParams(use_tc_tiling_on_sc=False,
                                             needs_layout_passes=False),
        out_type=jax.ShapeDtypeStruct((_NT, _EMBED_DIM, _CHUNK), jnp.float32),
        scratch_types=[
            pltpu.VMEM((_NCHUNK, _CHUNK), jnp.int32),
            pltpu.VMEM((_NCHUNK, _EMBED_DIM, _CHUNK), jnp.float32),
            pltpu.SemaphoreType.DMA,
        ],
    )
    def gather_kernel(table_hbm, idx_hbm, out_hbm, idx_v, slab_v, sem):
        wid = lax.axis_index("s") * _NC + lax.axis_index("c")
        pltpu.sync_copy(idx_hbm.at[wid], idx_v)

        def fire(j, carry):
            for c in range(_NCHUNK):
                pltpu.make_async_copy(
                    table_hbm.at[j].at[idx_v.at[c]],
                    slab_v.at[c, j],
                    sem,
                ).start()
            return carry

        lax.fori_loop(0, _EMBED_DIM, fire, 0)
        pltpu.make_async_copy(
            out_hbm.at[pl.ds(wid * _NCHUNK, _NCHUNK)], slab_v, sem).wait()
        pltpu.sync_copy(slab_v, out_hbm.at[pl.ds(wid * _NCHUNK, _NCHUNK)])

    return gather_kernel(tableT, idx3)


_BC = 2048
_TPB = _BC // 128


def _mlp1_body(featT_ref, w1T_ref, b1c_ref, xf_ref):
    w1fT = w1T_ref[:, _EMBED_DIM:]
    xf = jnp.dot(w1fT, featT_ref[...],
                 preferred_element_type=jnp.float32) + b1c_ref[...]
    xf_ref[...] = xf.astype(jnp.bfloat16)


def _tc_mlp1(featT, w1T, b1c):
    grid = (_BATCH // _BC,)
    return pl.pallas_call(
        _mlp1_body,
        grid=grid,
        in_specs=[
            pl.BlockSpec((_INPUT_DIM, _BC), lambda i: (0, i)),
            pl.BlockSpec((_INPUT_DIM, _EMBED_DIM + _INPUT_DIM),
                         lambda i: (0, 0)),
            pl.BlockSpec((_HIDDEN, 1), lambda i: (0, 0)),
        ],
        out_specs=pl.BlockSpec((_HIDDEN, _BC), lambda i: (0, i)),
        out_shape=jax.ShapeDtypeStruct((_HIDDEN, _BATCH), jnp.bfloat16),
    )(featT, w1T, b1c)


def _mlp2_body(xf_ref, e_ref, w1T_ref, w2c_ref, b2_ref, out_ref):
    w1eT = w1T_ref[:, 0:_EMBED_DIM]
    e = e_ref[...]
    xe = jnp.concatenate(
        [jnp.dot(w1eT, e[t], preferred_element_type=jnp.float32)
         for t in range(_TPB)], axis=1)
    hT = jnp.maximum(xf_ref[...].astype(jnp.float32) + xe, 0.0)
    o = jnp.sum(hT * w2c_ref[...], axis=0) + b2_ref[0, 0]
    out_ref[...] = jnp.concatenate(
        [o.reshape(1, _BC), jnp.zeros((7, _BC), jnp.float32)], axis=0)


def _tc_mlp2(xfT, E, w1T, w2c, b2r):
    grid = (_BATCH // _BC,)
    return pl.pallas_call(
        _mlp2_body,
        grid=grid,
        in_specs=[
            pl.BlockSpec((_HIDDEN, _BC), lambda i: (0, i)),
            pl.BlockSpec((_TPB, _EMBED_DIM, _CHUNK), lambda i: (i, 0, 0)),
            pl.BlockSpec((_INPUT_DIM, _EMBED_DIM + _INPUT_DIM),
                         lambda i: (0, 0)),
            pl.BlockSpec((_HIDDEN, 1), lambda i: (0, 0)),
            pl.BlockSpec((1, 1), lambda i: (0, 0)),
        ],
        out_specs=pl.BlockSpec((8, _BC), lambda i: (0, i)),
        out_shape=jax.ShapeDtypeStruct((8, _BATCH), jnp.float32),
    )(xfT, E, w1T, w2c, b2r)


def kernel(pitcher_id, features, table, W1, b1, W2, b2):
    pid = pitcher_id.astype(jnp.int32)
    idx3 = pid.reshape(_NW, _NCHUNK, _CHUNK)
    w1T = W1.T
    E = _sc_gather(table.T, idx3)
    xfT = _tc_mlp1(features.T, w1T, b1.reshape(_HIDDEN, 1))
    out8 = _tc_mlp2(xfT, E, w1T, W2, b2.reshape(1, 1))
    return out8[:1, :].reshape(_BATCH, 1)


# speedup vs baseline: 2.4368x; 1.0311x over previous
import functools

import jax
import jax.numpy as jnp
from jax import lax
from jax.experimental import pallas as pl
from jax.experimental.pallas import tpu as pltpu
from jax.experimental.pallas import tpu_sc as plsc

_EMBED_DIM = 16
_INPUT_DIM = 64
_HIDDEN = 64
_BATCH = 16384
_NT = _BATCH // 128

_NC = 2
_NS = 16
_NW = _NC * _NS
_BPW = _BATCH // _NW
_CHUNK = 128
_NCHUNK = _BPW // _CHUNK


def _sc_gather(tableT, idx3):
    mesh = plsc.VectorSubcoreMesh(core_axis_name="c", subcore_axis_name="s")

    @functools.partial(
        pl.kernel,
        mesh=mesh,
        compiler_params=pltpu.CompilerParams(use_tc_tiling_on_sc=False,
                                             needs_layout_passes=False),
        out_type=jax.ShapeDtypeStruct((_NT, _EMBED_DIM, _CHUNK), jnp.float32),
        scratch_types=[
            pltpu.VMEM((_NCHUNK, _CHUNK), jnp.int32),
            pltpu.VMEM((_NCHUNK, _EMBED_DIM, _CHUNK), jnp.float32),
            pltpu.SemaphoreType.DMA,
        ],
    )
    def gather_kernel(table_hbm, idx_hbm, out_hbm, idx_v, slab_v, sem):
        wid = lax.axis_index("s") * _NC + lax.axis_index("c")
        pltpu.sync_copy(idx_hbm.at[wid], idx_v)

        def fire(j, carry):
            for c in range(_NCHUNK):
                pltpu.make_async_copy(
                    table_hbm.at[j].at[idx_v.at[c]],
                    slab_v.at[c, j],
                    sem,
                ).start()
            return carry

        lax.fori_loop(0, _EMBED_DIM, fire, 0)
        pltpu.make_async_copy(
            out_hbm.at[pl.ds(wid * _NCHUNK, _NCHUNK)], slab_v, sem).wait()
        pltpu.sync_copy(slab_v, out_hbm.at[pl.ds(wid * _NCHUNK, _NCHUNK)])

    return gather_kernel(tableT, idx3)


_BC = 4096
_TPB = _BC // 128


def _mlp1_body(featT_ref, w1T_ref, b1c_ref, xf_ref):
    w1fT = w1T_ref[:, _EMBED_DIM:]
    xf = jnp.dot(w1fT, featT_ref[...],
                 preferred_element_type=jnp.float32) + b1c_ref[...]
    xf_ref[...] = xf.astype(jnp.bfloat16)


def _tc_mlp1(featT, w1T, b1c):
    grid = (_BATCH // _BC,)
    return pl.pallas_call(
        _mlp1_body,
        grid=grid,
        in_specs=[
            pl.BlockSpec((_INPUT_DIM, _BC), lambda i: (0, i)),
            pl.BlockSpec((_INPUT_DIM, _EMBED_DIM + _INPUT_DIM),
                         lambda i: (0, 0)),
            pl.BlockSpec((_HIDDEN, 1), lambda i: (0, 0)),
        ],
        out_specs=pl.BlockSpec((_HIDDEN, _BC), lambda i: (0, i)),
        out_shape=jax.ShapeDtypeStruct((_HIDDEN, _BATCH), jnp.bfloat16),
    )(featT, w1T, b1c)


def _mlp2_body(xf_ref, e_ref, w1T_ref, w2c_ref, b2_ref, out_ref):
    w1eT = w1T_ref[:, 0:_EMBED_DIM]
    e = e_ref[...]
    xe = jnp.concatenate(
        [jnp.dot(w1eT, e[t], preferred_element_type=jnp.float32)
         for t in range(_TPB)], axis=1)
    hT = jnp.maximum(xf_ref[...].astype(jnp.float32) + xe, 0.0)
    o = jnp.sum(hT * w2c_ref[...], axis=0) + b2_ref[0, 0]
    out_ref[...] = jnp.concatenate(
        [o.reshape(1, _BC), jnp.zeros((7, _BC), jnp.float32)], axis=0)


def _tc_mlp2(xfT, E, w1T, w2c, b2r):
    grid = (_BATCH // _BC,)
    return pl.pallas_call(
        _mlp2_body,
        grid=grid,
        in_specs=[
            pl.BlockSpec((_HIDDEN, _BC), lambda i: (0, i)),
            pl.BlockSpec((_TPB, _EMBED_DIM, _CHUNK), lambda i: (i, 0, 0)),
            pl.BlockSpec((_INPUT_DIM, _EMBED_DIM + _INPUT_DIM),
                         lambda i: (0, 0)),
            pl.BlockSpec((_HIDDEN, 1), lambda i: (0, 0)),
            pl.BlockSpec((1, 1), lambda i: (0, 0)),
        ],
        out_specs=pl.BlockSpec((8, _BC), lambda i: (0, i)),
        out_shape=jax.ShapeDtypeStruct((8, _BATCH), jnp.float32),
    )(xfT, E, w1T, w2c, b2r)


def kernel(pitcher_id, features, table, W1, b1, W2, b2):
    pid = pitcher_id.astype(jnp.int32)
    idx3 = pid.reshape(_NW, _NCHUNK, _CHUNK)
    w1T = W1.T
    E = _sc_gather(table.T, idx3)
    xfT = _tc_mlp1(features.T, w1T, b1.reshape(_HIDDEN, 1))
    out8 = _tc_mlp2(xfT, E, w1T, W2, b2.reshape(1, 1))
    return out8[:1, :].reshape(_BATCH, 1)
